# Initial kernel scaffold; baseline (speedup 1.0000x reference)
import jax
import jax.numpy as jnp
from jax.experimental import pallas as pl


def kernel(vertex_features, vertex_positions, img_feat0, img_feat1, img_feat2, img_feat3, lin0_w, gc0_w0, gc0_w1, gc1_w0, gc1_w1, gc2_w0, gc2_w1, lin1_w, vertex_adjacency):
    raise NotImplementedError("write your pallas kernel here")


# speedup vs baseline: 3.4522x; 3.4522x over previous
import functools

import jax
import jax.numpy as jnp
from jax import lax
from jax.experimental import pallas as pl
from jax.experimental.pallas import tpu as pltpu
from jax.experimental.pallas import tpu_sc as plsc

B = 4
NV = 2466
N = B * NV
E = 59184
NF = 128
NDIMS = 3
IMG_HW = 224

FEAT = [(256, 56), (512, 28), (1024, 14), (2048, 7)]
HWS = [s * s for _, s in FEAT]

N_PAD = 9984
VPT = 312
VCH = 104
NVCH = VPT // VCH

EPT = 2048
ECH = 128
NECH = EPT // ECH
E_PAD = 32 * EPT

ROWS_PT = N_PAD // 32
ACC_PT = N_PAD // 16

@functools.lru_cache(maxsize=None)
def _mesh():
    return plsc.VectorSubcoreMesh(
        core_axis_name="c", subcore_axis_name="s",
        num_cores=2, num_subcores=16)


def _dot(a, b):
    return lax.dot_general(a, b, (((1,), (0,)), ((), ())),
                           preferred_element_type=jnp.float32)


def _coords_body(px_ref, py_ref, pz_ref, out_ref):
    pxv = px_ref[...]
    pyv = py_ref[...]
    pzv = pz_ref[...]
    rows = lax.broadcasted_iota(jnp.int32, (N_PAD // 128, 128), 0)
    cols = lax.broadcasted_iota(jnp.int32, (N_PAD // 128, 128), 1)
    n = rows * 128 + cols
    valid_n = n < N
    pz_safe = jnp.where(valid_n, pzv, 1.0)
    h = 248.0 * (pyv / pz_safe) + 111.5
    w = 248.0 * (pxv / (-pz_safe)) + 111.5
    h = jnp.clip(h, 0.0, IMG_HW - 1.0)
    w = jnp.clip(w, 0.0, IMG_HW - 1.0)
    bidx = n // NV
    for m, (_, s) in enumerate(FEAT):
        x = w / (float(IMG_HW) / s)
        y = h / (float(IMG_HW) / s)
        x1 = jnp.floor(x).astype(jnp.int32)
        x2 = jnp.minimum(jnp.ceil(x), s - 1).astype(jnp.int32)
        y1 = jnp.floor(y).astype(jnp.int32)
        y2 = jnp.minimum(jnp.ceil(y), s - 1).astype(jnp.int32)
        ok = (x2 > x1) & (y2 > y1) & valid_n
        flat = bidx * (s * s) + x1 * s + y1
        out_ref[m, :, :] = jnp.where(ok, flat, B * s * s)


def _coords(px, py, pz):
    r = N_PAD // 128
    return pl.pallas_call(
        _coords_body,
        out_shape=jax.ShapeDtypeStruct((4, r, 128), jnp.int32),
    )(px, py, pz)


def _ptable_body(fm_ref, w_ref, out_ref):
    out_ref[0] = lax.dot_general(
        fm_ref[0], w_ref[...], (((0,), (0,)), ((), ())),
        preferred_element_type=jnp.float32)


def _ptable(fm, w, hw):
    c = fm.shape[1]
    out = pl.pallas_call(
        _ptable_body,
        grid=(B,),
        in_specs=[
            pl.BlockSpec((1, c, hw), lambda b: (b, 0, 0)),
            pl.BlockSpec((c, NF), lambda b: (0, 0)),
        ],
        out_specs=pl.BlockSpec((1, hw, NF), lambda b: (b, 0, 0)),
        out_shape=jax.ShapeDtypeStruct((B, hw, NF), jnp.float32),
    )(fm, w)
    return out.reshape(B * hw, NF)


_MBLK = 2496


def _conv_in_body(vf_ref, pos_ref, pj_ref,
                  w0a, w0b, w0c, w1a, w1b, w1c, o0_ref, o1_ref):
    vf = vf_ref[...]
    ps = pos_ref[...]
    pj = pj_ref[0] + pj_ref[1] + pj_ref[2] + pj_ref[3]
    o0_ref[...] = _dot(vf, w0a[...]) + _dot(ps, w0b[...]) + _dot(pj, w0c[...])
    o1_ref[...] = _dot(vf, w1a[...]) + _dot(ps, w1b[...]) + _dot(pj, w1c[...])


def _conv_in(vfeat, pos, pj, w0, w1):
    w0a, w0b, w0c = w0[:NF], w0[NF:NF + NDIMS], w0[NF + NDIMS:]
    w1a, w1b, w1c = w1[:NF], w1[NF:NF + NDIMS], w1[NF + NDIMS:]
    g = N_PAD // _MBLK
    row = lambda i: (i, 0)
    full = lambda i: (0, 0)
    return pl.pallas_call(
        _conv_in_body,
        grid=(g,),
        in_specs=[
            pl.BlockSpec((_MBLK, NF), row),
            pl.BlockSpec((_MBLK, NDIMS), row),
            pl.BlockSpec((4, _MBLK, NF), lambda i: (0, i, 0)),
            pl.BlockSpec((NF, NF), full),
            pl.BlockSpec((NDIMS, NF), full),
            pl.BlockSpec((NF, NF), full),
            pl.BlockSpec((NF, NF), full),
            pl.BlockSpec((NDIMS, NF), full),
            pl.BlockSpec((NF, NF), full),
        ],
        out_specs=[pl.BlockSpec((_MBLK, NF), row),
                   pl.BlockSpec((_MBLK, NF), row)],
        out_shape=[jax.ShapeDtypeStruct((N_PAD, NF), jnp.float32),
                   jax.ShapeDtypeStruct((N_PAD, NF), jnp.float32)],
    )(vfeat, pos, pj, w0a, w0b, w0c, w1a, w1b, w1c)


def _conv_mid_body(pos_ref, a_ref, part_ref,
                   w0a, w0b, w1a, w1b, o0_ref, o1_ref):
    ps = pos_ref[...]
    h = jnp.maximum(a_ref[...] + part_ref[0] + part_ref[1], 0.0)
    o0_ref[...] = _dot(ps, w0a[...]) + _dot(h, w0b[...])
    o1_ref[...] = _dot(ps, w1a[...]) + _dot(h, w1b[...])


def _conv_mid(pos, a, partials, w0, w1):
    w0a, w0b = w0[:NDIMS], w0[NDIMS:]
    w1a, w1b = w1[:NDIMS], w1[NDIMS:]
    g = N_PAD // _MBLK
    row = lambda i: (i, 0)
    full = lambda i: (0, 0)
    return pl.pallas_call(
        _conv_mid_body,
        grid=(g,),
        in_specs=[
            pl.BlockSpec((_MBLK, NDIMS), row),
            pl.BlockSpec((_MBLK, NF), row),
            pl.BlockSpec((2, _MBLK, NF), lambda i: (0, i, 0)),
            pl.BlockSpec((NDIMS, NF), full),
            pl.BlockSpec((NF, NF), full),
            pl.BlockSpec((NDIMS, NF), full),
            pl.BlockSpec((NF, NF), full),
        ],
        out_specs=[pl.BlockSpec((_MBLK, NF), row),
                   pl.BlockSpec((_MBLK, NF), row)],
        out_shape=[jax.ShapeDtypeStruct((N_PAD, NF), jnp.float32),
                   jax.ShapeDtypeStruct((N_PAD, NF), jnp.float32)],
    )(pos, a, partials, w0a, w0b, w1a, w1b)


def _final_body(pos_ref, a_ref, part_ref, lw_ref, nf_ref, np_ref):
    nf = jnp.maximum(a_ref[...] + part_ref[0] + part_ref[1], 0.0)
    nf_ref[...] = nf
    np_ref[...] = pos_ref[...] + jnp.tanh(_dot(nf, lw_ref[...]))


def _final(pos, a, partials, lin1_w):
    g = N_PAD // _MBLK
    row = lambda i: (i, 0)
    return pl.pallas_call(
        _final_body,
        grid=(g,),
        in_specs=[
            pl.BlockSpec((_MBLK, NDIMS), row),
            pl.BlockSpec((_MBLK, NF), row),
            pl.BlockSpec((2, _MBLK, NF), lambda i: (0, i, 0)),
            pl.BlockSpec((NF, NDIMS), lambda i: (0, 0)),
        ],
        out_specs=[pl.BlockSpec((_MBLK, NF), row),
                   pl.BlockSpec((_MBLK, NDIMS), row)],
        out_shape=[jax.ShapeDtypeStruct((N_PAD, NF), jnp.float32),
                   jax.ShapeDtypeStruct((N_PAD, NDIMS), jnp.float32)],
    )(pos, a, partials, lin1_w)


def _align_body(i0, i1, i2, i3, p0, p1, p2, p3, out_hbm, idxb, g, sem):
    wid = lax.axis_index("s") * 2 + lax.axis_index("c")
    base = wid * ROWS_PT
    for m, (im, pm) in enumerate(zip((i0, i1, i2, i3), (p0, p1, p2, p3))):
        for ch in range(NVCH):
            off = base + ch * VCH
            pltpu.sync_copy(im.at[pl.ds(off, VCH)], idxb)
            pltpu.async_copy(pm.at[idxb], g, sem).wait()
            pltpu.sync_copy(g, out_hbm.at[m, pl.ds(off, VCH)])


@functools.lru_cache(maxsize=None)
def _sc_align_fn():
    return pl.kernel(
        _align_body,
        out_type=jax.ShapeDtypeStruct((4, N_PAD, NF), jnp.float32),
        mesh=_mesh(),
        scratch_types=[
            pltpu.VMEM((VCH,), jnp.int32),
            pltpu.VMEM((VCH, NF), jnp.float32),
            pltpu.SemaphoreType.DMA,
        ],
    )


def _sc_align(idx, p0, p1, p2, p3):
    return _sc_align_fn()(idx[0], idx[1], idx[2], idx[3], p0, p1, p2, p3)


def _seg_body(src_hbm, dst_hbm, xw1_hbm, zeros_hbm, out_hbm,
              srcv, dstv, g, acc, sem):
    c = lax.axis_index("c")
    s = lax.axis_index("s")
    wid = s * 2 + c
    pltpu.sync_copy(zeros_hbm, acc.at[pl.ds(s * ACC_PT, ACC_PT)])
    pltpu.sync_copy(src_hbm.at[wid], srcv)
    pltpu.sync_copy(dst_hbm.at[wid], dstv)
    plsc.subcore_barrier()
    for k in range(NECH):
        pltpu.async_copy(xw1_hbm.at[srcv.at[k]], g, sem).wait()
        pltpu.sync_copy(g, acc.at[dstv.at[k]], add=True)
    plsc.subcore_barrier()
    pltpu.sync_copy(acc.at[pl.ds(s * ACC_PT, ACC_PT)],
                    out_hbm.at[c, pl.ds(s * ACC_PT, ACC_PT)])


@functools.lru_cache(maxsize=None)
def _sc_segsum_fn():
    return pl.kernel(
        _seg_body,
        out_type=jax.ShapeDtypeStruct((2, N_PAD, NF), jnp.float32),
        mesh=_mesh(),
        scratch_types=[
            pltpu.VMEM((NECH, ECH), jnp.int32),
            pltpu.VMEM((NECH, ECH), jnp.int32),
            pltpu.VMEM((ECH, NF), jnp.float32),
            pltpu.VMEM_SHARED((N_PAD, NF), jnp.float32),
            pltpu.SemaphoreType.DMA,
        ],
    )


def _sc_segsum(src, dst, xw1, zeros):
    return _sc_segsum_fn()(src, dst, xw1, zeros)


def kernel(vertex_features, vertex_positions, img_feat0, img_feat1,
           img_feat2, img_feat3, lin0_w, gc0_w0, gc0_w1, gc1_w0, gc1_w1,
           gc2_w0, gc2_w1, lin1_w, vertex_adjacency):
    f32 = jnp.float32
    pad_n = N_PAD - N

    pos = jnp.pad(vertex_positions, ((0, pad_n), (0, 0)))
    vfeat = jnp.pad(vertex_features, ((0, pad_n), (0, 0)))

    r = N_PAD // 128
    px = pos[:, 0].reshape(r, 128)
    py = pos[:, 1].reshape(r, 128)
    pz = pos[:, 2].reshape(r, 128)
    idx = _coords(px, py, pz).reshape(4, N_PAD)

    fms = [img_feat0, img_feat1, img_feat2, img_feat3]
    ptables = []
    off = 0
    for m, (c, s) in enumerate(FEAT):
        w_m = lin0_w[off:off + c]
        off += c
        p = _ptable(fms[m].reshape(B, c, s * s), w_m, s * s)
        ptables.append(jnp.pad(p, ((0, 8), (0, 0))))

    proj = _sc_align(idx, *ptables)

    src = jnp.concatenate(
        [vertex_adjacency[0],
         jnp.full((E_PAD - E,), N, jnp.int32)]).reshape(32, NECH, ECH)
    dst = jnp.concatenate(
        [vertex_adjacency[1],
         jnp.zeros((E_PAD - E,), jnp.int32)]).reshape(32, NECH, ECH)
    zeros = jnp.zeros((ACC_PT, NF), f32)

    xw0, xw1 = _conv_in(vfeat, pos, proj, gc0_w0, gc0_w1)
    part = _sc_segsum(src, dst, xw1, zeros)
    xw0, xw1 = _conv_mid(pos, xw0, part, gc1_w0, gc1_w1)
    part = _sc_segsum(src, dst, xw1, zeros)
    xw0, xw1 = _conv_mid(pos, xw0, part, gc2_w0, gc2_w1)
    part = _sc_segsum(src, dst, xw1, zeros)
    nf, new_pos = _final(pos, xw0, part, lin1_w)

    return (new_pos[:N], nf[:N])


# speedup vs baseline: 3.6189x; 1.0483x over previous
import functools

import jax
import jax.numpy as jnp
from jax import lax
from jax.experimental import pallas as pl
from jax.experimental.pallas import tpu as pltpu
from jax.experimental.pallas import tpu_sc as plsc

B = 4
NV = 2466
N = B * NV
E = 59184
NF = 128
NDIMS = 3
IMG_HW = 224

FEAT = [(256, 56), (512, 28), (1024, 14), (2048, 7)]
HWS = [s * s for _, s in FEAT]

N_PAD = 9984
VPT = 312
VCH = 104
NVCH = VPT // VCH

EPT = 2048
ECH = 128
NECH = EPT // ECH
E_PAD = 32 * EPT

ROWS_PT = N_PAD // 32
ACC_PT = N_PAD // 16

@functools.lru_cache(maxsize=None)
def _mesh():
    return plsc.VectorSubcoreMesh(
        core_axis_name="c", subcore_axis_name="s",
        num_cores=2, num_subcores=16)


def _dot(a, b):
    return lax.dot_general(a, b, (((1,), (0,)), ((), ())),
                           preferred_element_type=jnp.float32)


def _coords_body(px_ref, py_ref, pz_ref, out_ref):
    pxv = px_ref[...]
    pyv = py_ref[...]
    pzv = pz_ref[...]
    rows = lax.broadcasted_iota(jnp.int32, (N_PAD // 128, 128), 0)
    cols = lax.broadcasted_iota(jnp.int32, (N_PAD // 128, 128), 1)
    n = rows * 128 + cols
    valid_n = n < N
    pz_safe = jnp.where(valid_n, pzv, 1.0)
    h = 248.0 * (pyv / pz_safe) + 111.5
    w = 248.0 * (pxv / (-pz_safe)) + 111.5
    h = jnp.clip(h, 0.0, IMG_HW - 1.0)
    w = jnp.clip(w, 0.0, IMG_HW - 1.0)
    bidx = n // NV
    for m, (_, s) in enumerate(FEAT):
        x = w / (float(IMG_HW) / s)
        y = h / (float(IMG_HW) / s)
        x1 = jnp.floor(x).astype(jnp.int32)
        x2 = jnp.minimum(jnp.ceil(x), s - 1).astype(jnp.int32)
        y1 = jnp.floor(y).astype(jnp.int32)
        y2 = jnp.minimum(jnp.ceil(y), s - 1).astype(jnp.int32)
        ok = (x2 > x1) & (y2 > y1) & valid_n
        flat = bidx * (s * s) + x1 * s + y1
        out_ref[m, :, :] = jnp.where(ok, flat, B * s * s)


def _coords(px, py, pz):
    r = N_PAD // 128
    return pl.pallas_call(
        _coords_body,
        out_shape=jax.ShapeDtypeStruct((4, r, 128), jnp.int32),
    )(px, py, pz)


def _ptable_body(fm_ref, w_ref, out_ref):
    out_ref[0] = lax.dot_general(
        fm_ref[0], w_ref[...], (((0,), (0,)), ((), ())),
        preferred_element_type=jnp.float32)


def _ptable(fm, w, hw):
    c = fm.shape[1]
    out = pl.pallas_call(
        _ptable_body,
        grid=(B,),
        in_specs=[
            pl.BlockSpec((1, c, hw), lambda b: (b, 0, 0)),
            pl.BlockSpec((c, NF), lambda b: (0, 0)),
        ],
        out_specs=pl.BlockSpec((1, hw, NF), lambda b: (b, 0, 0)),
        out_shape=jax.ShapeDtypeStruct((B, hw, NF), jnp.float32),
    )(fm, w)
    return out.reshape(B * hw, NF)


_MBLK = 2496


def _conv_in_body(vf_ref, pos_ref, pj_ref,
                  w0a, w0b, w0c, w1a, w1b, w1c, o0_ref, o1_ref):
    vf = vf_ref[...]
    ps = pos_ref[...]
    pj = pj_ref[0] + pj_ref[1] + pj_ref[2] + pj_ref[3]
    o0_ref[...] = _dot(vf, w0a[...]) + _dot(ps, w0b[...]) + _dot(pj, w0c[...])
    o1_ref[...] = _dot(vf, w1a[...]) + _dot(ps, w1b[...]) + _dot(pj, w1c[...])


def _conv_in(vfeat, pos, pj, w0, w1):
    w0a, w0b, w0c = w0[:NF], w0[NF:NF + NDIMS], w0[NF + NDIMS:]
    w1a, w1b, w1c = w1[:NF], w1[NF:NF + NDIMS], w1[NF + NDIMS:]
    g = N_PAD // _MBLK
    row = lambda i: (i, 0)
    full = lambda i: (0, 0)
    return pl.pallas_call(
        _conv_in_body,
        grid=(g,),
        in_specs=[
            pl.BlockSpec((_MBLK, NF), row),
            pl.BlockSpec((_MBLK, NDIMS), row),
            pl.BlockSpec((4, _MBLK, NF), lambda i: (0, i, 0)),
            pl.BlockSpec((NF, NF), full),
            pl.BlockSpec((NDIMS, NF), full),
            pl.BlockSpec((NF, NF), full),
            pl.BlockSpec((NF, NF), full),
            pl.BlockSpec((NDIMS, NF), full),
            pl.BlockSpec((NF, NF), full),
        ],
        out_specs=[pl.BlockSpec((_MBLK, NF), row),
                   pl.BlockSpec((_MBLK, NF), row)],
        out_shape=[jax.ShapeDtypeStruct((N_PAD, NF), jnp.float32),
                   jax.ShapeDtypeStruct((N_PAD, NF), jnp.float32)],
    )(vfeat, pos, pj, w0a, w0b, w0c, w1a, w1b, w1c)


def _conv_mid_body(pos_ref, a_ref, part_ref,
                   w0a, w0b, w1a, w1b, o0_ref, o1_ref):
    ps = pos_ref[...]
    h = jnp.maximum(a_ref[...] + part_ref[0] + part_ref[1], 0.0)
    o0_ref[...] = _dot(ps, w0a[...]) + _dot(h, w0b[...])
    o1_ref[...] = _dot(ps, w1a[...]) + _dot(h, w1b[...])


def _conv_mid(pos, a, partials, w0, w1):
    w0a, w0b = w0[:NDIMS], w0[NDIMS:]
    w1a, w1b = w1[:NDIMS], w1[NDIMS:]
    g = N_PAD // _MBLK
    row = lambda i: (i, 0)
    full = lambda i: (0, 0)
    return pl.pallas_call(
        _conv_mid_body,
        grid=(g,),
        in_specs=[
            pl.BlockSpec((_MBLK, NDIMS), row),
            pl.BlockSpec((_MBLK, NF), row),
            pl.BlockSpec((2, _MBLK, NF), lambda i: (0, i, 0)),
            pl.BlockSpec((NDIMS, NF), full),
            pl.BlockSpec((NF, NF), full),
            pl.BlockSpec((NDIMS, NF), full),
            pl.BlockSpec((NF, NF), full),
        ],
        out_specs=[pl.BlockSpec((_MBLK, NF), row),
                   pl.BlockSpec((_MBLK, NF), row)],
        out_shape=[jax.ShapeDtypeStruct((N_PAD, NF), jnp.float32),
                   jax.ShapeDtypeStruct((N_PAD, NF), jnp.float32)],
    )(pos, a, partials, w0a, w0b, w1a, w1b)


def _final_body(pos_ref, a_ref, part_ref, lw_ref, nf_ref, np_ref):
    nf = jnp.maximum(a_ref[...] + part_ref[0] + part_ref[1], 0.0)
    nf_ref[...] = nf
    np_ref[...] = pos_ref[...] + jnp.tanh(_dot(nf, lw_ref[...]))


def _final(pos, a, partials, lin1_w):
    g = N_PAD // _MBLK
    row = lambda i: (i, 0)
    return pl.pallas_call(
        _final_body,
        grid=(g,),
        in_specs=[
            pl.BlockSpec((_MBLK, NDIMS), row),
            pl.BlockSpec((_MBLK, NF), row),
            pl.BlockSpec((2, _MBLK, NF), lambda i: (0, i, 0)),
            pl.BlockSpec((NF, NDIMS), lambda i: (0, 0)),
        ],
        out_specs=[pl.BlockSpec((_MBLK, NF), row),
                   pl.BlockSpec((_MBLK, NDIMS), row)],
        out_shape=[jax.ShapeDtypeStruct((N_PAD, NF), jnp.float32),
                   jax.ShapeDtypeStruct((N_PAD, NDIMS), jnp.float32)],
    )(pos, a, partials, lin1_w)


_NB = 4
_NB_S = 2


def _align_body(i0, i1, i2, i3, p0, p1, p2, p3, out_hbm,
                iv0, iv1, iv2, iv3, g0, g1, g2, g3, gs0, gs1, gs2, gs3,
                os0, os1, os2, os3):
    wid = lax.axis_index("s") * 2 + lax.axis_index("c")
    base = wid * ROWS_PT
    gs = (g0, g1, g2, g3)
    gsem = (gs0, gs1, gs2, gs3)
    osem = (os0, os1, os2, os3)
    ptabs = (p0, p1, p2, p3)
    idxv = (iv0, iv1, iv2, iv3)
    for m, im in enumerate((i0, i1, i2, i3)):
        pltpu.sync_copy(im.at[pl.ds(base, ROWS_PT)], idxv[m])
    tasks = [(m, ch) for m in range(4) for ch in range(NVCH)]
    nt = len(tasks)

    def fire(t, b):
        m, ch = tasks[t]
        idx = idxv[m].at[pl.ds(ch * VCH, VCH)]
        return pltpu.async_copy(ptabs[m].at[idx], gs[b], gsem[b])

    gcp = [fire(t, t) for t in range(_NB)]
    ocp = [None] * _NB
    for t in range(nt):
        b = t % _NB
        m, ch = tasks[t]
        gcp[b].wait()
        off = base + ch * VCH
        ocp[b] = pltpu.async_copy(gs[b], out_hbm.at[m, pl.ds(off, VCH)],
                                  osem[b])
        if t + _NB < nt:
            ocp[b].wait()
            gcp[b] = fire(t + _NB, b)
    for t in range(max(0, nt - _NB), nt):
        ocp[t % _NB].wait()


@functools.lru_cache(maxsize=None)
def _sc_align_fn():
    return pl.kernel(
        _align_body,
        out_type=jax.ShapeDtypeStruct((4, N_PAD, NF), jnp.float32),
        mesh=_mesh(),
        scratch_types=(
            [pltpu.VMEM((ROWS_PT,), jnp.int32)] * 4
            + [pltpu.VMEM((VCH, NF), jnp.float32)] * _NB
            + [pltpu.SemaphoreType.DMA] * (2 * _NB)
        ),
    )


def _sc_align(idx, p0, p1, p2, p3):
    return _sc_align_fn()(idx[0], idx[1], idx[2], idx[3], p0, p1, p2, p3)


def _seg_body(src_hbm, dst_hbm, xw1_hbm, zeros_hbm, out_hbm,
              srcv, dstv, g0, g1, acc,
              gs0, gs1, zsem):
    c = lax.axis_index("c")
    s = lax.axis_index("s")
    wid = s * 2 + c
    gs = (g0, g1)
    gsem = (gs0, gs1)
    zcp = pltpu.async_copy(zeros_hbm, acc.at[pl.ds(s * ACC_PT, ACC_PT)], zsem)
    pltpu.sync_copy(src_hbm.at[wid], srcv)
    pltpu.sync_copy(dst_hbm.at[wid], dstv)

    def fire(k, b):
        return pltpu.async_copy(xw1_hbm.at[srcv.at[k]], gs[b], gsem[b])

    gcp = [fire(k, k) for k in range(_NB_S)]
    zcp.wait()
    plsc.subcore_barrier()
    for k in range(NECH):
        b = k % _NB_S
        gcp[b].wait()
        pltpu.sync_copy(gs[b], acc.at[dstv.at[k]], add=True)
        if k + _NB_S < NECH:
            gcp[b] = fire(k + _NB_S, b)
    plsc.subcore_barrier()
    pltpu.sync_copy(acc.at[pl.ds(s * ACC_PT, ACC_PT)],
                    out_hbm.at[c, pl.ds(s * ACC_PT, ACC_PT)])


@functools.lru_cache(maxsize=None)
def _sc_segsum_fn():
    return pl.kernel(
        _seg_body,
        out_type=jax.ShapeDtypeStruct((2, N_PAD, NF), jnp.float32),
        mesh=_mesh(),
        scratch_types=(
            [pltpu.VMEM((NECH, ECH), jnp.int32),
             pltpu.VMEM((NECH, ECH), jnp.int32)]
            + [pltpu.VMEM((ECH, NF), jnp.float32)] * _NB_S
            + [pltpu.VMEM_SHARED((N_PAD, NF), jnp.float32)]
            + [pltpu.SemaphoreType.DMA] * (_NB_S + 1)
        ),
    )


def _sc_segsum(src, dst, xw1, zeros):
    return _sc_segsum_fn()(src, dst, xw1, zeros)


def kernel(vertex_features, vertex_positions, img_feat0, img_feat1,
           img_feat2, img_feat3, lin0_w, gc0_w0, gc0_w1, gc1_w0, gc1_w1,
           gc2_w0, gc2_w1, lin1_w, vertex_adjacency):
    f32 = jnp.float32
    pad_n = N_PAD - N

    pos = jnp.pad(vertex_positions, ((0, pad_n), (0, 0)))
    vfeat = jnp.pad(vertex_features, ((0, pad_n), (0, 0)))

    r = N_PAD // 128
    px = pos[:, 0].reshape(r, 128)
    py = pos[:, 1].reshape(r, 128)
    pz = pos[:, 2].reshape(r, 128)
    idx = _coords(px, py, pz).reshape(4, N_PAD)

    fms = [img_feat0, img_feat1, img_feat2, img_feat3]
    ptables = []
    off = 0
    for m, (c, s) in enumerate(FEAT):
        w_m = lin0_w[off:off + c]
        off += c
        p = _ptable(fms[m].reshape(B, c, s * s), w_m, s * s)
        ptables.append(jnp.pad(p, ((0, 8), (0, 0))))

    proj = _sc_align(idx, *ptables)

    src = jnp.concatenate(
        [vertex_adjacency[0],
         jnp.full((E_PAD - E,), N, jnp.int32)]).reshape(32, NECH, ECH)
    dst = jnp.concatenate(
        [vertex_adjacency[1],
         jnp.zeros((E_PAD - E,), jnp.int32)]).reshape(32, NECH, ECH)
    zeros = jnp.zeros((ACC_PT, NF), f32)

    xw0, xw1 = _conv_in(vfeat, pos, proj, gc0_w0, gc0_w1)
    part = _sc_segsum(src, dst, xw1, zeros)
    xw0, xw1 = _conv_mid(pos, xw0, part, gc1_w0, gc1_w1)
    part = _sc_segsum(src, dst, xw1, zeros)
    xw0, xw1 = _conv_mid(pos, xw0, part, gc2_w0, gc2_w1)
    part = _sc_segsum(src, dst, xw1, zeros)
    nf, new_pos = _final(pos, xw0, part, lin1_w)

    return (new_pos[:N], nf[:N])


# speedup vs baseline: 3.6722x; 1.0147x over previous
import functools

import jax
import jax.numpy as jnp
from jax import lax
from jax.experimental import pallas as pl
from jax.experimental.pallas import tpu as pltpu
from jax.experimental.pallas import tpu_sc as plsc

B = 4
NV = 2466
N = B * NV
E = 59184
NF = 128
NDIMS = 3
IMG_HW = 224

FEAT = [(256, 56), (512, 28), (1024, 14), (2048, 7)]
HWS = [s * s for _, s in FEAT]

N_PAD = 9984
VPT = 312
VCH = 104
NVCH = VPT // VCH

EPT = 2048
ECH = 64
NECH = EPT // ECH
E_PAD = 32 * EPT

ROWS_PT = N_PAD // 32
ACC_PT = N_PAD // 16

@functools.lru_cache(maxsize=None)
def _mesh():
    return plsc.VectorSubcoreMesh(
        core_axis_name="c", subcore_axis_name="s",
        num_cores=2, num_subcores=16)


def _dot(a, b):
    return lax.dot_general(a, b, (((1,), (0,)), ((), ())),
                           preferred_element_type=jnp.float32)


def _coords_body(px_ref, py_ref, pz_ref, out_ref):
    pxv = px_ref[...]
    pyv = py_ref[...]
    pzv = pz_ref[...]
    rows = lax.broadcasted_iota(jnp.int32, (N_PAD // 128, 128), 0)
    cols = lax.broadcasted_iota(jnp.int32, (N_PAD // 128, 128), 1)
    n = rows * 128 + cols
    valid_n = n < N
    pz_safe = jnp.where(valid_n, pzv, 1.0)
    h = 248.0 * (pyv / pz_safe) + 111.5
    w = 248.0 * (pxv / (-pz_safe)) + 111.5
    h = jnp.clip(h, 0.0, IMG_HW - 1.0)
    w = jnp.clip(w, 0.0, IMG_HW - 1.0)
    bidx = n // NV
    for m, (_, s) in enumerate(FEAT):
        x = w / (float(IMG_HW) / s)
        y = h / (float(IMG_HW) / s)
        x1 = jnp.floor(x).astype(jnp.int32)
        x2 = jnp.minimum(jnp.ceil(x), s - 1).astype(jnp.int32)
        y1 = jnp.floor(y).astype(jnp.int32)
        y2 = jnp.minimum(jnp.ceil(y), s - 1).astype(jnp.int32)
        ok = (x2 > x1) & (y2 > y1) & valid_n
        flat = bidx * (s * s) + x1 * s + y1
        out_ref[m, :, :] = jnp.where(ok, flat, B * s * s)


def _coords(px, py, pz):
    r = N_PAD // 128
    return pl.pallas_call(
        _coords_body,
        out_shape=jax.ShapeDtypeStruct((4, r, 128), jnp.int32),
    )(px, py, pz)


def _ptable_body(fm_ref, w_ref, out_ref):
    out_ref[0] = lax.dot_general(
        fm_ref[0], w_ref[...], (((0,), (0,)), ((), ())),
        preferred_element_type=jnp.float32)


def _ptable(fm, w, hw):
    c = fm.shape[1]
    out = pl.pallas_call(
        _ptable_body,
        grid=(B,),
        in_specs=[
            pl.BlockSpec((1, c, hw), lambda b: (b, 0, 0)),
            pl.BlockSpec((c, NF), lambda b: (0, 0)),
        ],
        out_specs=pl.BlockSpec((1, hw, NF), lambda b: (b, 0, 0)),
        out_shape=jax.ShapeDtypeStruct((B, hw, NF), jnp.float32),
    )(fm, w)
    return out.reshape(B * hw, NF)


_MBLK = 2496


def _conv_in_body(vf_ref, pos_ref, pj_ref,
                  w0a, w0b, w0c, w1a, w1b, w1c, o0_ref, o1_ref):
    vf = vf_ref[...]
    ps = pos_ref[...]
    pj = pj_ref[0] + pj_ref[1] + pj_ref[2] + pj_ref[3]
    o0_ref[...] = _dot(vf, w0a[...]) + _dot(ps, w0b[...]) + _dot(pj, w0c[...])
    o1_ref[...] = _dot(vf, w1a[...]) + _dot(ps, w1b[...]) + _dot(pj, w1c[...])


def _conv_in(vfeat, pos, pj, w0, w1):
    w0a, w0b, w0c = w0[:NF], w0[NF:NF + NDIMS], w0[NF + NDIMS:]
    w1a, w1b, w1c = w1[:NF], w1[NF:NF + NDIMS], w1[NF + NDIMS:]
    g = N_PAD // _MBLK
    row = lambda i: (i, 0)
    full = lambda i: (0, 0)
    return pl.pallas_call(
        _conv_in_body,
        grid=(g,),
        in_specs=[
            pl.BlockSpec((_MBLK, NF), row),
            pl.BlockSpec((_MBLK, NDIMS), row),
            pl.BlockSpec((4, _MBLK, NF), lambda i: (0, i, 0)),
            pl.BlockSpec((NF, NF), full),
            pl.BlockSpec((NDIMS, NF), full),
            pl.BlockSpec((NF, NF), full),
            pl.BlockSpec((NF, NF), full),
            pl.BlockSpec((NDIMS, NF), full),
            pl.BlockSpec((NF, NF), full),
        ],
        out_specs=[pl.BlockSpec((_MBLK, NF), row),
                   pl.BlockSpec((_MBLK, NF), row)],
        out_shape=[jax.ShapeDtypeStruct((N_PAD, NF), jnp.float32),
                   jax.ShapeDtypeStruct((N_PAD, NF), jnp.float32)],
    )(vfeat, pos, pj, w0a, w0b, w0c, w1a, w1b, w1c)


def _conv_mid_body(pos_ref, a_ref, part_ref,
                   w0a, w0b, w1a, w1b, o0_ref, o1_ref):
    ps = pos_ref[...]
    h = jnp.maximum(a_ref[...] + part_ref[0] + part_ref[1], 0.0)
    o0_ref[...] = _dot(ps, w0a[...]) + _dot(h, w0b[...])
    o1_ref[...] = _dot(ps, w1a[...]) + _dot(h, w1b[...])


def _conv_mid(pos, a, partials, w0, w1):
    w0a, w0b = w0[:NDIMS], w0[NDIMS:]
    w1a, w1b = w1[:NDIMS], w1[NDIMS:]
    g = N_PAD // _MBLK
    row = lambda i: (i, 0)
    full = lambda i: (0, 0)
    return pl.pallas_call(
        _conv_mid_body,
        grid=(g,),
        in_specs=[
            pl.BlockSpec((_MBLK, NDIMS), row),
            pl.BlockSpec((_MBLK, NF), row),
            pl.BlockSpec((2, _MBLK, NF), lambda i: (0, i, 0)),
            pl.BlockSpec((NDIMS, NF), full),
            pl.BlockSpec((NF, NF), full),
            pl.BlockSpec((NDIMS, NF), full),
            pl.BlockSpec((NF, NF), full),
        ],
        out_specs=[pl.BlockSpec((_MBLK, NF), row),
                   pl.BlockSpec((_MBLK, NF), row)],
        out_shape=[jax.ShapeDtypeStruct((N_PAD, NF), jnp.float32),
                   jax.ShapeDtypeStruct((N_PAD, NF), jnp.float32)],
    )(pos, a, partials, w0a, w0b, w1a, w1b)


def _final_body(pos_ref, a_ref, part_ref, lw_ref, nf_ref, np_ref):
    nf = jnp.maximum(a_ref[...] + part_ref[0] + part_ref[1], 0.0)
    nf_ref[...] = nf
    np_ref[...] = pos_ref[...] + jnp.tanh(_dot(nf, lw_ref[...]))


def _final(pos, a, partials, lin1_w):
    g = N_PAD // _MBLK
    row = lambda i: (i, 0)
    return pl.pallas_call(
        _final_body,
        grid=(g,),
        in_specs=[
            pl.BlockSpec((_MBLK, NDIMS), row),
            pl.BlockSpec((_MBLK, NF), row),
            pl.BlockSpec((2, _MBLK, NF), lambda i: (0, i, 0)),
            pl.BlockSpec((NF, NDIMS), lambda i: (0, 0)),
        ],
        out_specs=[pl.BlockSpec((_MBLK, NF), row),
                   pl.BlockSpec((_MBLK, NDIMS), row)],
        out_shape=[jax.ShapeDtypeStruct((N_PAD, NF), jnp.float32),
                   jax.ShapeDtypeStruct((N_PAD, NDIMS), jnp.float32)],
    )(pos, a, partials, lin1_w)


_NB = 8
_NB_S = 5


def _align_body(i0, i1, i2, i3, p0, p1, p2, p3, out_hbm,
                iv0, iv1, iv2, iv3, g0, g1, g2, g3, g4, g5, g6, g7,
                gs0, gs1, gs2, gs3, gs4, gs5, gs6, gs7,
                os0, os1, os2, os3, os4, os5, os6, os7):
    wid = lax.axis_index("s") * 2 + lax.axis_index("c")
    base = wid * ROWS_PT
    gs = (g0, g1, g2, g3, g4, g5, g6, g7)
    gsem = (gs0, gs1, gs2, gs3, gs4, gs5, gs6, gs7)
    osem = (os0, os1, os2, os3, os4, os5, os6, os7)
    ptabs = (p0, p1, p2, p3)
    idxv = (iv0, iv1, iv2, iv3)
    for m, im in enumerate((i0, i1, i2, i3)):
        pltpu.sync_copy(im.at[pl.ds(base, ROWS_PT)], idxv[m])
    tasks = [(m, ch) for m in range(4) for ch in range(NVCH)]
    nt = len(tasks)

    def fire(t, b):
        m, ch = tasks[t]
        idx = idxv[m].at[pl.ds(ch * VCH, VCH)]
        return pltpu.async_copy(ptabs[m].at[idx], gs[b], gsem[b])

    gcp = [fire(t, t) for t in range(_NB)]
    ocp = [None] * _NB
    for t in range(nt):
        b = t % _NB
        m, ch = tasks[t]
        gcp[b].wait()
        off = base + ch * VCH
        ocp[b] = pltpu.async_copy(gs[b], out_hbm.at[m, pl.ds(off, VCH)],
                                  osem[b])
        if t + _NB < nt:
            ocp[b].wait()
            gcp[b] = fire(t + _NB, b)
    for t in range(max(0, nt - _NB), nt):
        ocp[t % _NB].wait()


@functools.lru_cache(maxsize=None)
def _sc_align_fn():
    return pl.kernel(
        _align_body,
        out_type=jax.ShapeDtypeStruct((4, N_PAD, NF), jnp.float32),
        mesh=_mesh(),
        scratch_types=(
            [pltpu.VMEM((ROWS_PT,), jnp.int32)] * 4
            + [pltpu.VMEM((VCH, NF), jnp.float32)] * _NB
            + [pltpu.SemaphoreType.DMA] * (2 * _NB)
        ),
    )


def _sc_align(idx, p0, p1, p2, p3):
    return _sc_align_fn()(idx[0], idx[1], idx[2], idx[3], p0, p1, p2, p3)


def _seg_body(src_hbm, dst_hbm, xw1_hbm, zeros_hbm, out_hbm,
              srcv, dstv, g0, g1, g2, g3, g4, acc,
              gs0, gs1, gs2, gs3, gs4, zsem):
    c = lax.axis_index("c")
    s = lax.axis_index("s")
    wid = s * 2 + c
    gs = (g0, g1, g2, g3, g4)
    gsem = (gs0, gs1, gs2, gs3, gs4)
    zcp = pltpu.async_copy(zeros_hbm, acc.at[pl.ds(s * ACC_PT, ACC_PT)], zsem)
    pltpu.sync_copy(src_hbm.at[wid], srcv)
    pltpu.sync_copy(dst_hbm.at[wid], dstv)

    def fire(k, b):
        return pltpu.async_copy(xw1_hbm.at[srcv.at[k]], gs[b], gsem[b])

    gcp = [fire(k, k) for k in range(_NB_S)]
    zcp.wait()
    plsc.subcore_barrier()
    for k in range(NECH):
        b = k % _NB_S
        gcp[b].wait()
        pltpu.sync_copy(gs[b], acc.at[dstv.at[k]], add=True)
        if k + _NB_S < NECH:
            gcp[b] = fire(k + _NB_S, b)
    plsc.subcore_barrier()
    pltpu.sync_copy(acc.at[pl.ds(s * ACC_PT, ACC_PT)],
                    out_hbm.at[c, pl.ds(s * ACC_PT, ACC_PT)])


@functools.lru_cache(maxsize=None)
def _sc_segsum_fn():
    return pl.kernel(
        _seg_body,
        out_type=jax.ShapeDtypeStruct((2, N_PAD, NF), jnp.float32),
        mesh=_mesh(),
        scratch_types=(
            [pltpu.VMEM((NECH, ECH), jnp.int32),
             pltpu.VMEM((NECH, ECH), jnp.int32)]
            + [pltpu.VMEM((ECH, NF), jnp.float32)] * _NB_S
            + [pltpu.VMEM_SHARED((N_PAD, NF), jnp.float32)]
            + [pltpu.SemaphoreType.DMA] * (_NB_S + 1)
        ),
    )


def _sc_segsum(src, dst, xw1, zeros):
    return _sc_segsum_fn()(src, dst, xw1, zeros)


def kernel(vertex_features, vertex_positions, img_feat0, img_feat1,
           img_feat2, img_feat3, lin0_w, gc0_w0, gc0_w1, gc1_w0, gc1_w1,
           gc2_w0, gc2_w1, lin1_w, vertex_adjacency):
    f32 = jnp.float32
    pad_n = N_PAD - N

    pos = jnp.pad(vertex_positions, ((0, pad_n), (0, 0)))
    vfeat = jnp.pad(vertex_features, ((0, pad_n), (0, 0)))

    r = N_PAD // 128
    px = pos[:, 0].reshape(r, 128)
    py = pos[:, 1].reshape(r, 128)
    pz = pos[:, 2].reshape(r, 128)
    idx = _coords(px, py, pz).reshape(4, N_PAD)

    fms = [img_feat0, img_feat1, img_feat2, img_feat3]
    ptables = []
    off = 0
    for m, (c, s) in enumerate(FEAT):
        w_m = lin0_w[off:off + c]
        off += c
        p = _ptable(fms[m].reshape(B, c, s * s), w_m, s * s)
        ptables.append(jnp.pad(p, ((0, 8), (0, 0))))

    proj = _sc_align(idx, *ptables)

    src = jnp.concatenate(
        [vertex_adjacency[0],
         jnp.full((E_PAD - E,), N, jnp.int32)]).reshape(32, NECH, ECH)
    dst = jnp.concatenate(
        [vertex_adjacency[1],
         jnp.zeros((E_PAD - E,), jnp.int32)]).reshape(32, NECH, ECH)
    zeros = jnp.zeros((ACC_PT, NF), f32)

    xw0, xw1 = _conv_in(vfeat, pos, proj, gc0_w0, gc0_w1)
    part = _sc_segsum(src, dst, xw1, zeros)
    xw0, xw1 = _conv_mid(pos, xw0, part, gc1_w0, gc1_w1)
    part = _sc_segsum(src, dst, xw1, zeros)
    xw0, xw1 = _conv_mid(pos, xw0, part, gc2_w0, gc2_w1)
    part = _sc_segsum(src, dst, xw1, zeros)
    nf, new_pos = _final(pos, xw0, part, lin1_w)

    return (new_pos[:N], nf[:N])


# speedup vs baseline: 6.9436x; 1.8909x over previous
import functools

import jax
import jax.numpy as jnp
from jax import lax
from jax.experimental import pallas as pl
from jax.experimental.pallas import tpu as pltpu
from jax.experimental.pallas import tpu_sc as plsc

B = 4
NV = 2466
N = B * NV
E = 59184
NF = 128
NDIMS = 3
IMG_HW = 224

FEAT = [(256, 56), (512, 28), (1024, 14), (2048, 7)]
HWS = [s * s for _, s in FEAT]

N_PAD = 9984
VPT = 312
VCH = 104
NVCH = VPT // VCH

E_PAD = 65536
HALF = 4992
ECH = 32
CPS = 16
NSEG = 8
ACC_R = 9872
DUMP = N

ROWS_PT = N_PAD // 32
ACC_PT = N_PAD // 16

@functools.lru_cache(maxsize=None)
def _mesh():
    return plsc.VectorSubcoreMesh(
        core_axis_name="c", subcore_axis_name="s",
        num_cores=2, num_subcores=16)


def _dot(a, b):
    return lax.dot_general(a, b, (((1,), (0,)), ((), ())),
                           preferred_element_type=jnp.float32)


def _coords_body(px_ref, py_ref, pz_ref, out_ref):
    pxv = px_ref[...]
    pyv = py_ref[...]
    pzv = pz_ref[...]
    rows = lax.broadcasted_iota(jnp.int32, (N_PAD // 128, 128), 0)
    cols = lax.broadcasted_iota(jnp.int32, (N_PAD // 128, 128), 1)
    n = rows * 128 + cols
    valid_n = n < N
    pz_safe = jnp.where(valid_n, pzv, 1.0)
    h = 248.0 * (pyv / pz_safe) + 111.5
    w = 248.0 * (pxv / (-pz_safe)) + 111.5
    h = jnp.clip(h, 0.0, IMG_HW - 1.0)
    w = jnp.clip(w, 0.0, IMG_HW - 1.0)
    bidx = n // NV
    for m, (_, s) in enumerate(FEAT):
        x = w / (float(IMG_HW) / s)
        y = h / (float(IMG_HW) / s)
        x1 = jnp.floor(x).astype(jnp.int32)
        x2 = jnp.minimum(jnp.ceil(x), s - 1).astype(jnp.int32)
        y1 = jnp.floor(y).astype(jnp.int32)
        y2 = jnp.minimum(jnp.ceil(y), s - 1).astype(jnp.int32)
        ok = (x2 > x1) & (y2 > y1) & valid_n
        flat = bidx * (s * s) + x1 * s + y1
        out_ref[m, :, :] = jnp.where(ok, flat, B * s * s)


def _coords(px, py, pz):
    r = N_PAD // 128
    return pl.pallas_call(
        _coords_body,
        out_shape=jax.ShapeDtypeStruct((4, r, 128), jnp.int32),
    )(px, py, pz)


def _ptable_body(fm_ref, w_ref, out_ref):
    out_ref[0] = lax.dot_general(
        fm_ref[0], w_ref[...], (((0,), (0,)), ((), ())),
        preferred_element_type=jnp.float32)


def _ptable(fm, w, hw):
    c = fm.shape[1]
    out = pl.pallas_call(
        _ptable_body,
        grid=(B,),
        in_specs=[
            pl.BlockSpec((1, c, hw), lambda b: (b, 0, 0)),
            pl.BlockSpec((c, NF), lambda b: (0, 0)),
        ],
        out_specs=pl.BlockSpec((1, hw, NF), lambda b: (b, 0, 0)),
        out_shape=jax.ShapeDtypeStruct((B, hw, NF), jnp.float32),
    )(fm, w)
    return out.reshape(B * hw, NF)


_MBLK = 2496


def _conv_in_body(vf_ref, pos_ref, pj_ref,
                  w0a, w0b, w0c, w1a, w1b, w1c, o0_ref, o1_ref):
    vf = vf_ref[...]
    ps = pos_ref[...]
    pj = pj_ref[0] + pj_ref[1] + pj_ref[2] + pj_ref[3]
    o0_ref[...] = _dot(vf, w0a[...]) + _dot(ps, w0b[...]) + _dot(pj, w0c[...])
    o1_ref[...] = _dot(vf, w1a[...]) + _dot(ps, w1b[...]) + _dot(pj, w1c[...])


def _conv_in(vfeat, pos, pj, w0, w1):
    w0a, w0b, w0c = w0[:NF], w0[NF:NF + NDIMS], w0[NF + NDIMS:]
    w1a, w1b, w1c = w1[:NF], w1[NF:NF + NDIMS], w1[NF + NDIMS:]
    g = N_PAD // _MBLK
    row = lambda i: (i, 0)
    full = lambda i: (0, 0)
    return pl.pallas_call(
        _conv_in_body,
        grid=(g,),
        in_specs=[
            pl.BlockSpec((_MBLK, NF), row),
            pl.BlockSpec((_MBLK, NDIMS), row),
            pl.BlockSpec((4, _MBLK, NF), lambda i: (0, i, 0)),
            pl.BlockSpec((NF, NF), full),
            pl.BlockSpec((NDIMS, NF), full),
            pl.BlockSpec((NF, NF), full),
            pl.BlockSpec((NF, NF), full),
            pl.BlockSpec((NDIMS, NF), full),
            pl.BlockSpec((NF, NF), full),
        ],
        out_specs=[pl.BlockSpec((_MBLK, NF), row),
                   pl.BlockSpec((_MBLK, NF), row)],
        out_shape=[jax.ShapeDtypeStruct((N_PAD, NF), jnp.float32),
                   jax.ShapeDtypeStruct((N_PAD, NF), jnp.float32)],
    )(vfeat, pos, pj, w0a, w0b, w0c, w1a, w1b, w1c)


def _conv_mid_body(pos_ref, a_ref, part_ref,
                   w0a, w0b, w1a, w1b, o0_ref, o1_ref):
    ps = pos_ref[...]
    h = jnp.maximum(a_ref[...] + part_ref[0] + part_ref[1], 0.0)
    o0_ref[...] = _dot(ps, w0a[...]) + _dot(h, w0b[...])
    o1_ref[...] = _dot(ps, w1a[...]) + _dot(h, w1b[...])


def _conv_mid(pos, a, partials, w0, w1):
    w0a, w0b = w0[:NDIMS], w0[NDIMS:]
    w1a, w1b = w1[:NDIMS], w1[NDIMS:]
    g = N_PAD // _MBLK
    row = lambda i: (i, 0)
    full = lambda i: (0, 0)
    return pl.pallas_call(
        _conv_mid_body,
        grid=(g,),
        in_specs=[
            pl.BlockSpec((_MBLK, NDIMS), row),
            pl.BlockSpec((_MBLK, NF), row),
            pl.BlockSpec((2, _MBLK, NF), lambda i: (0, i, 0)),
            pl.BlockSpec((NDIMS, NF), full),
            pl.BlockSpec((NF, NF), full),
            pl.BlockSpec((NDIMS, NF), full),
            pl.BlockSpec((NF, NF), full),
        ],
        out_specs=[pl.BlockSpec((_MBLK, NF), row),
                   pl.BlockSpec((_MBLK, NF), row)],
        out_shape=[jax.ShapeDtypeStruct((N_PAD, NF), jnp.float32),
                   jax.ShapeDtypeStruct((N_PAD, NF), jnp.float32)],
    )(pos, a, partials, w0a, w0b, w1a, w1b)


def _final_body(pos_ref, a_ref, part_ref, lw_ref, nf_ref, np_ref):
    nf = jnp.maximum(a_ref[...] + part_ref[0] + part_ref[1], 0.0)
    nf_ref[...] = nf
    np_ref[...] = pos_ref[...] + jnp.tanh(_dot(nf, lw_ref[...]))


def _final(pos, a, partials, lin1_w):
    g = N_PAD // _MBLK
    row = lambda i: (i, 0)
    return pl.pallas_call(
        _final_body,
        grid=(g,),
        in_specs=[
            pl.BlockSpec((_MBLK, NDIMS), row),
            pl.BlockSpec((_MBLK, NF), row),
            pl.BlockSpec((2, _MBLK, NF), lambda i: (0, i, 0)),
            pl.BlockSpec((NF, NDIMS), lambda i: (0, 0)),
        ],
        out_specs=[pl.BlockSpec((_MBLK, NF), row),
                   pl.BlockSpec((_MBLK, NDIMS), row)],
        out_shape=[jax.ShapeDtypeStruct((N_PAD, NF), jnp.float32),
                   jax.ShapeDtypeStruct((N_PAD, NDIMS), jnp.float32)],
    )(pos, a, partials, lin1_w)


_NB = 8
_NB_S = 5


def _align_body(i0, i1, i2, i3, p0, p1, p2, p3, out_hbm,
                iv0, iv1, iv2, iv3, g0, g1, g2, g3, g4, g5, g6, g7,
                gs0, gs1, gs2, gs3, gs4, gs5, gs6, gs7,
                os0, os1, os2, os3, os4, os5, os6, os7):
    wid = lax.axis_index("s") * 2 + lax.axis_index("c")
    base = wid * ROWS_PT
    gs = (g0, g1, g2, g3, g4, g5, g6, g7)
    gsem = (gs0, gs1, gs2, gs3, gs4, gs5, gs6, gs7)
    osem = (os0, os1, os2, os3, os4, os5, os6, os7)
    ptabs = (p0, p1, p2, p3)
    idxv = (iv0, iv1, iv2, iv3)
    for m, im in enumerate((i0, i1, i2, i3)):
        pltpu.sync_copy(im.at[pl.ds(base, ROWS_PT)], idxv[m])
    tasks = [(m, ch) for m in range(4) for ch in range(NVCH)]
    nt = len(tasks)

    def fire(t, b):
        m, ch = tasks[t]
        idx = idxv[m].at[pl.ds(ch * VCH, VCH)]
        return pltpu.async_copy(ptabs[m].at[idx], gs[b], gsem[b])

    gcp = [fire(t, t) for t in range(_NB)]
    ocp = [None] * _NB
    for t in range(nt):
        b = t % _NB
        m, ch = tasks[t]
        gcp[b].wait()
        off = base + ch * VCH
        ocp[b] = pltpu.async_copy(gs[b], out_hbm.at[m, pl.ds(off, VCH)],
                                  osem[b])
        if t + _NB < nt:
            ocp[b].wait()
            gcp[b] = fire(t + _NB, b)
    for t in range(max(0, nt - _NB), nt):
        ocp[t % _NB].wait()


@functools.lru_cache(maxsize=None)
def _sc_align_fn():
    return pl.kernel(
        _align_body,
        out_type=jax.ShapeDtypeStruct((4, N_PAD, NF), jnp.float32),
        mesh=_mesh(),
        scratch_types=(
            [pltpu.VMEM((ROWS_PT,), jnp.int32)] * 4
            + [pltpu.VMEM((VCH, NF), jnp.float32)] * _NB
            + [pltpu.SemaphoreType.DMA] * (2 * _NB)
        ),
    )


def _sc_align(idx, p0, p1, p2, p3):
    return _sc_align_fn()(idx[0], idx[1], idx[2], idx[3], p0, p1, p2, p3)


def _seg_body(srcm_hbm, dstm_hbm, xw1_hbm, zeros_hbm, out_hbm,
              srcv, dstv, g0, g1, xw1s, acc, gs0, gs1, ssem):
    c = lax.axis_index("c")
    s = lax.axis_index("s")
    gs = (g0, g1)
    gsem = (gs0, gs1)
    scp = pltpu.async_copy(xw1_hbm.at[pl.ds(c * HALF + s * ROWS_PT, ROWS_PT)],
                           xw1s.at[pl.ds(s * ROWS_PT, ROWS_PT)], ssem)
    @pl.when(s < 15)
    def _():
        pltpu.sync_copy(zeros_hbm, acc.at[pl.ds(s * ACC_PT, ACC_PT)])

    @pl.when(s == 15)
    def _():
        pltpu.sync_copy(zeros_hbm.at[pl.ds(0, ACC_R - 15 * ACC_PT)],
                        acc.at[pl.ds(15 * ACC_PT, ACC_R - 15 * ACC_PT)])

    scp.wait()
    plsc.subcore_barrier()
    for t in range(NSEG):
        seg = (c * 16 + s) * NSEG + t
        pltpu.sync_copy(srcm_hbm.at[seg], srcv)
        pltpu.sync_copy(dstm_hbm.at[seg], dstv)

        def fire(j, b):
            idx = srcv.at[j // 4, pl.ds((j % 4) * ECH, ECH)]
            return pltpu.async_copy(xw1s.at[idx], gs[b], gsem[b])

        gcp = [fire(j, j) for j in range(2)]
        for j in range(CPS):
            b = j % 2
            gcp[b].wait()
            pltpu.sync_copy(gs[b], acc.at[dstv.at[j]], add=True)
            if j + 2 < CPS:
                gcp[b] = fire(j + 2, b)
    plsc.subcore_barrier()
    @pl.when(s < 15)
    def _():
        pltpu.sync_copy(acc.at[pl.ds(s * ACC_PT, ACC_PT)],
                        out_hbm.at[c, pl.ds(s * ACC_PT, ACC_PT)])

    @pl.when(s == 15)
    def _():
        pltpu.sync_copy(acc.at[pl.ds(15 * ACC_PT, N - 15 * ACC_PT)],
                        out_hbm.at[c, pl.ds(15 * ACC_PT, N - 15 * ACC_PT)])
        pltpu.sync_copy(zeros_hbm.at[pl.ds(0, N_PAD - N)],
                        out_hbm.at[c, pl.ds(N, N_PAD - N)])


@functools.lru_cache(maxsize=None)
def _sc_segsum_fn():
    return pl.kernel(
        _seg_body,
        out_type=jax.ShapeDtypeStruct((2, N_PAD, NF), jnp.float32),
        mesh=_mesh(),
        scratch_types=(
            [pltpu.VMEM((4, 128), jnp.int32),
             pltpu.VMEM((CPS, ECH), jnp.int32),
             pltpu.VMEM((ECH, NF), jnp.float32),
             pltpu.VMEM((ECH, NF), jnp.float32),
             pltpu.VMEM_SHARED((HALF, NF), jnp.float32),
             pltpu.VMEM_SHARED((ACC_R, NF), jnp.float32)]
            + [pltpu.SemaphoreType.DMA] * 3
        ),
    )


def _sc_segsum(src, dst, xw1, zeros):
    return _sc_segsum_fn()(src, dst, xw1, zeros)


def _edgemap_body(src_ref, dst_ref, sm_ref, dm_ref):
    sv = src_ref[...]
    dv = dst_ref[...]
    in0 = sv < HALF
    sm_ref[0] = jnp.where(in0, sv, 0)
    sm_ref[1] = jnp.where(in0, 0, sv - HALF)
    dm_ref[0] = jnp.where(in0, dv, DUMP)
    dm_ref[1] = jnp.where(in0, DUMP, dv)


def _edgemap(src, dst):
    r = E_PAD // 128
    return pl.pallas_call(
        _edgemap_body,
        out_shape=[jax.ShapeDtypeStruct((2, r, 128), jnp.int32),
                   jax.ShapeDtypeStruct((2, r, 128), jnp.int32)],
    )(src.reshape(r, 128), dst.reshape(r, 128))


def kernel(vertex_features, vertex_positions, img_feat0, img_feat1,
           img_feat2, img_feat3, lin0_w, gc0_w0, gc0_w1, gc1_w0, gc1_w1,
           gc2_w0, gc2_w1, lin1_w, vertex_adjacency):
    f32 = jnp.float32
    pad_n = N_PAD - N

    pos = jnp.pad(vertex_positions, ((0, pad_n), (0, 0)))
    vfeat = jnp.pad(vertex_features, ((0, pad_n), (0, 0)))

    r = N_PAD // 128
    px = pos[:, 0].reshape(r, 128)
    py = pos[:, 1].reshape(r, 128)
    pz = pos[:, 2].reshape(r, 128)
    idx = _coords(px, py, pz).reshape(4, N_PAD)

    fms = [img_feat0, img_feat1, img_feat2, img_feat3]
    ptables = []
    off = 0
    for m, (c, s) in enumerate(FEAT):
        w_m = lin0_w[off:off + c]
        off += c
        p = _ptable(fms[m].reshape(B, c, s * s), w_m, s * s)
        ptables.append(jnp.pad(p, ((0, 8), (0, 0))))

    proj = _sc_align(idx, *ptables)

    src = jnp.concatenate([vertex_adjacency[0],
                           jnp.full((E_PAD - E,), N, jnp.int32)])
    dst = jnp.concatenate([vertex_adjacency[1],
                           jnp.zeros((E_PAD - E,), jnp.int32)])
    srcm, dstm = _edgemap(src, dst)
    srcm = srcm.reshape(2 * 16 * NSEG, 4, 128)
    dstm = dstm.reshape(2 * 16 * NSEG, CPS, ECH)
    zeros = jnp.zeros((ACC_PT, NF), f32)

    xw0, xw1 = _conv_in(vfeat, pos, proj, gc0_w0, gc0_w1)
    part = _sc_segsum(srcm, dstm, xw1, zeros)
    xw0, xw1 = _conv_mid(pos, xw0, part, gc1_w0, gc1_w1)
    part = _sc_segsum(srcm, dstm, xw1, zeros)
    xw0, xw1 = _conv_mid(pos, xw0, part, gc2_w0, gc2_w1)
    part = _sc_segsum(srcm, dstm, xw1, zeros)
    nf, new_pos = _final(pos, xw0, part, lin1_w)

    return (new_pos[:N], nf[:N])


# speedup vs baseline: 9.5625x; 1.3772x over previous
import functools

import jax
import jax.numpy as jnp
from jax import lax
from jax.experimental import pallas as pl
from jax.experimental.pallas import tpu as pltpu
from jax.experimental.pallas import tpu_sc as plsc

B = 4
NV = 2466
N = B * NV
E = 59184
NF = 128
NDIMS = 3
IMG_HW = 224

FEAT = [(256, 56), (512, 28), (1024, 14), (2048, 7)]
HWS = [s * s for _, s in FEAT]

N_PAD = 9984
VPT = 624
VCH = 104
NVCH = VPT // VCH

E_PAD = 65536
HALF = 4992
ECH = 32
CPS = 16
NSEG = 8
ACC_R = 9872
DUMP = N

ROWS_PT = N_PAD // 32
ACC_PT = N_PAD // 16

M0H = 6280
STG_R = 9472
STG_PT = STG_R // 16
ZR0 = M0H + 3144
ZR1 = M0H + 792

@functools.lru_cache(maxsize=None)
def _mesh():
    return plsc.VectorSubcoreMesh(
        core_axis_name="c", subcore_axis_name="s",
        num_cores=2, num_subcores=16)


def _dot(a, b):
    return lax.dot_general(a, b, (((1,), (0,)), ((), ())),
                           preferred_element_type=jnp.float32)


def _coords_body(px_ref, py_ref, pz_ref, out_ref):
    pxv = px_ref[...]
    pyv = py_ref[...]
    pzv = pz_ref[...]
    rows = lax.broadcasted_iota(jnp.int32, (N_PAD // 128, 128), 0)
    cols = lax.broadcasted_iota(jnp.int32, (N_PAD // 128, 128), 1)
    n = rows * 128 + cols
    valid_n = n < N
    pz_safe = jnp.where(valid_n, pzv, 1.0)
    h = 248.0 * (pyv / pz_safe) + 111.5
    w = 248.0 * (pxv / (-pz_safe)) + 111.5
    h = jnp.clip(h, 0.0, IMG_HW - 1.0)
    w = jnp.clip(w, 0.0, IMG_HW - 1.0)
    bidx = n // NV
    gidx = []
    for m, (_, s) in enumerate(FEAT):
        x = w / (float(IMG_HW) / s)
        y = h / (float(IMG_HW) / s)
        x1 = jnp.floor(x).astype(jnp.int32)
        x2 = jnp.minimum(jnp.ceil(x), s - 1).astype(jnp.int32)
        y1 = jnp.floor(y).astype(jnp.int32)
        y2 = jnp.minimum(jnp.ceil(y), s - 1).astype(jnp.int32)
        ok = (x2 > x1) & (y2 > y1) & valid_n
        gidx.append((ok, bidx * (s * s) + x1 * s + y1))
    ok0, g0 = gidx[0]
    ok1, g1 = gidx[1]
    ok2, g2 = gidx[2]
    ok3, g3 = gidx[3]
    out_ref[0, 0] = jnp.where(ok0 & (g0 < M0H), g0, ZR0)
    out_ref[0, 1] = jnp.where(ok0 & (g0 >= M0H), g0 - M0H, ZR1)
    out_ref[1, 0] = jnp.where(ok1, M0H + g1, ZR0)
    out_ref[1, 1] = jnp.where(ok2, M0H + g2, ZR1)
    out_ref[2, 0] = jnp.full_like(g0, ZR0)
    out_ref[2, 1] = jnp.where(ok3, M0H + 800 + g3, ZR1)


def _coords(px, py, pz):
    r = N_PAD // 128
    return pl.pallas_call(
        _coords_body,
        out_shape=jax.ShapeDtypeStruct((3, 2, r, 128), jnp.int32),
    )(px, py, pz)


def _ptable_body(fm_ref, w_ref, out_ref):
    out_ref[0] = lax.dot_general(
        fm_ref[0], w_ref[...], (((0,), (0,)), ((), ())),
        preferred_element_type=jnp.float32)


def _ptable(fm, w, hw):
    c = fm.shape[1]
    out = pl.pallas_call(
        _ptable_body,
        grid=(B,),
        in_specs=[
            pl.BlockSpec((1, c, hw), lambda b: (b, 0, 0)),
            pl.BlockSpec((c, NF), lambda b: (0, 0)),
        ],
        out_specs=pl.BlockSpec((1, hw, NF), lambda b: (b, 0, 0)),
        out_shape=jax.ShapeDtypeStruct((B, hw, NF), jnp.float32),
    )(fm, w)
    return out.reshape(B * hw, NF)


_MBLK = 2496


def _conv_in_body(vf_ref, pos_ref, pj_ref,
                  w0a, w0b, w0c, w1a, w1b, w1c, o0_ref, o1_ref):
    vf = vf_ref[...]
    ps = pos_ref[...]
    pj = pj_ref[0] + pj_ref[1]
    o0_ref[...] = _dot(vf, w0a[...]) + _dot(ps, w0b[...]) + _dot(pj, w0c[...])
    o1_ref[...] = _dot(vf, w1a[...]) + _dot(ps, w1b[...]) + _dot(pj, w1c[...])


def _conv_in(vfeat, pos, pj, w0, w1):
    w0a, w0b, w0c = w0[:NF], w0[NF:NF + NDIMS], w0[NF + NDIMS:]
    w1a, w1b, w1c = w1[:NF], w1[NF:NF + NDIMS], w1[NF + NDIMS:]
    g = N_PAD // _MBLK
    row = lambda i: (i, 0)
    full = lambda i: (0, 0)
    return pl.pallas_call(
        _conv_in_body,
        grid=(g,),
        in_specs=[
            pl.BlockSpec((_MBLK, NF), row),
            pl.BlockSpec((_MBLK, NDIMS), row),
            pl.BlockSpec((2, _MBLK, NF), lambda i: (0, i, 0)),
            pl.BlockSpec((NF, NF), full),
            pl.BlockSpec((NDIMS, NF), full),
            pl.BlockSpec((NF, NF), full),
            pl.BlockSpec((NF, NF), full),
            pl.BlockSpec((NDIMS, NF), full),
            pl.BlockSpec((NF, NF), full),
        ],
        out_specs=[pl.BlockSpec((_MBLK, NF), row),
                   pl.BlockSpec((_MBLK, NF), row)],
        out_shape=[jax.ShapeDtypeStruct((N_PAD, NF), jnp.float32),
                   jax.ShapeDtypeStruct((N_PAD, NF), jnp.float32)],
    )(vfeat, pos, pj, w0a, w0b, w0c, w1a, w1b, w1c)


def _conv_mid_body(pos_ref, a_ref, part_ref,
                   w0a, w0b, w1a, w1b, o0_ref, o1_ref):
    ps = pos_ref[...]
    h = jnp.maximum(a_ref[...] + part_ref[0] + part_ref[1], 0.0)
    o0_ref[...] = _dot(ps, w0a[...]) + _dot(h, w0b[...])
    o1_ref[...] = _dot(ps, w1a[...]) + _dot(h, w1b[...])


def _conv_mid(pos, a, partials, w0, w1):
    w0a, w0b = w0[:NDIMS], w0[NDIMS:]
    w1a, w1b = w1[:NDIMS], w1[NDIMS:]
    g = N_PAD // _MBLK
    row = lambda i: (i, 0)
    full = lambda i: (0, 0)
    return pl.pallas_call(
        _conv_mid_body,
        grid=(g,),
        in_specs=[
            pl.BlockSpec((_MBLK, NDIMS), row),
            pl.BlockSpec((_MBLK, NF), row),
            pl.BlockSpec((2, _MBLK, NF), lambda i: (0, i, 0)),
            pl.BlockSpec((NDIMS, NF), full),
            pl.BlockSpec((NF, NF), full),
            pl.BlockSpec((NDIMS, NF), full),
            pl.BlockSpec((NF, NF), full),
        ],
        out_specs=[pl.BlockSpec((_MBLK, NF), row),
                   pl.BlockSpec((_MBLK, NF), row)],
        out_shape=[jax.ShapeDtypeStruct((N_PAD, NF), jnp.float32),
                   jax.ShapeDtypeStruct((N_PAD, NF), jnp.float32)],
    )(pos, a, partials, w0a, w0b, w1a, w1b)


def _final_body(pos_ref, a_ref, part_ref, lw_ref, nf_ref, np_ref):
    nf = jnp.maximum(a_ref[...] + part_ref[0] + part_ref[1], 0.0)
    nf_ref[...] = nf
    np_ref[...] = pos_ref[...] + jnp.tanh(_dot(nf, lw_ref[...]))


def _final(pos, a, partials, lin1_w):
    g = N_PAD // _MBLK
    row = lambda i: (i, 0)
    return pl.pallas_call(
        _final_body,
        grid=(g,),
        in_specs=[
            pl.BlockSpec((_MBLK, NDIMS), row),
            pl.BlockSpec((_MBLK, NF), row),
            pl.BlockSpec((2, _MBLK, NF), lambda i: (0, i, 0)),
            pl.BlockSpec((NF, NDIMS), lambda i: (0, 0)),
        ],
        out_specs=[pl.BlockSpec((_MBLK, NF), row),
                   pl.BlockSpec((_MBLK, NDIMS), row)],
        out_shape=[jax.ShapeDtypeStruct((N_PAD, NF), jnp.float32),
                   jax.ShapeDtypeStruct((N_PAD, NDIMS), jnp.float32)],
    )(pos, a, partials, lin1_w)


def _align_body(i0, i1, i2, stg_hbm, out_hbm,
                iv0, iv1, iv2, g0, g1, g2, stgv,
                gs0, gs1, gs2, ssem):
    c = lax.axis_index("c")
    s = lax.axis_index("s")
    base = s * VPT
    gs = (g0, g1, g2)
    gsem = (gs0, gs1, gs2)
    idxv = (iv0, iv1, iv2)
    scp = pltpu.async_copy(stg_hbm.at[c, pl.ds(s * STG_PT, STG_PT)],
                           stgv.at[pl.ds(s * STG_PT, STG_PT)], ssem)
    for m, im in enumerate((i0, i1, i2)):
        pltpu.sync_copy(im.at[pl.ds(c * N_PAD + base, VPT)], idxv[m])
    scp.wait()
    plsc.subcore_barrier()
    for ch in range(NVCH):
        off = base + ch * VCH
        cps = [pltpu.async_copy(
                   stgv.at[idxv[m].at[pl.ds(ch * VCH, VCH)]], gs[m], gsem[m])
               for m in range(3)]
        for cp in cps:
            cp.wait()

        def body(i, _):
            for k in range(NF // 16):
                sl = pl.ds(k * 16, 16)
                g0[i, sl] = g0[i, sl] + g1[i, sl] + g2[i, sl]
            return 0

        lax.fori_loop(0, VCH, body, 0, unroll=2)
        pltpu.sync_copy(g0, out_hbm.at[c, pl.ds(off, VCH)])


@functools.lru_cache(maxsize=None)
def _sc_align_fn():
    return pl.kernel(
        _align_body,
        out_type=jax.ShapeDtypeStruct((2, N_PAD, NF), jnp.float32),
        mesh=_mesh(),
        scratch_types=(
            [pltpu.VMEM((VPT,), jnp.int32)] * 3
            + [pltpu.VMEM((VCH, NF), jnp.float32)] * 3
            + [pltpu.VMEM_SHARED((STG_R, NF), jnp.float32)]
            + [pltpu.SemaphoreType.DMA] * 4
        ),
    )


def _sc_align(idx, stg):
    return _sc_align_fn()(idx[0], idx[1], idx[2], stg)


def _seg_body(srcm_hbm, dstm_hbm, xw1_hbm, zeros_hbm, out_hbm,
              srcv, dstv, g0, g1, xw1s, acc, gs0, gs1, ssem):
    c = lax.axis_index("c")
    s = lax.axis_index("s")
    gs = (g0, g1)
    gsem = (gs0, gs1)
    scp = pltpu.async_copy(xw1_hbm.at[pl.ds(c * HALF + s * ROWS_PT, ROWS_PT)],
                           xw1s.at[pl.ds(s * ROWS_PT, ROWS_PT)], ssem)
    @pl.when(s < 15)
    def _():
        pltpu.sync_copy(zeros_hbm, acc.at[pl.ds(s * ACC_PT, ACC_PT)])

    @pl.when(s == 15)
    def _():
        pltpu.sync_copy(zeros_hbm.at[pl.ds(0, ACC_R - 15 * ACC_PT)],
                        acc.at[pl.ds(15 * ACC_PT, ACC_R - 15 * ACC_PT)])

    scp.wait()
    plsc.subcore_barrier()
    for t in range(NSEG):
        seg = (c * 16 + s) * NSEG + t
        pltpu.sync_copy(srcm_hbm.at[seg], srcv)
        pltpu.sync_copy(dstm_hbm.at[seg], dstv)

        def fire(j, b):
            idx = srcv.at[j // 4, pl.ds((j % 4) * ECH, ECH)]
            return pltpu.async_copy(xw1s.at[idx], gs[b], gsem[b])

        gcp = [fire(j, j) for j in range(2)]
        for j in range(CPS):
            b = j % 2
            gcp[b].wait()
            pltpu.sync_copy(gs[b], acc.at[dstv.at[j]], add=True)
            if j + 2 < CPS:
                gcp[b] = fire(j + 2, b)
    plsc.subcore_barrier()
    @pl.when(s < 15)
    def _():
        pltpu.sync_copy(acc.at[pl.ds(s * ACC_PT, ACC_PT)],
                        out_hbm.at[c, pl.ds(s * ACC_PT, ACC_PT)])

    @pl.when(s == 15)
    def _():
        pltpu.sync_copy(acc.at[pl.ds(15 * ACC_PT, N - 15 * ACC_PT)],
                        out_hbm.at[c, pl.ds(15 * ACC_PT, N - 15 * ACC_PT)])
        pltpu.sync_copy(zeros_hbm.at[pl.ds(0, N_PAD - N)],
                        out_hbm.at[c, pl.ds(N, N_PAD - N)])


@functools.lru_cache(maxsize=None)
def _sc_segsum_fn():
    return pl.kernel(
        _seg_body,
        out_type=jax.ShapeDtypeStruct((2, N_PAD, NF), jnp.float32),
        mesh=_mesh(),
        scratch_types=(
            [pltpu.VMEM((4, 128), jnp.int32),
             pltpu.VMEM((CPS, ECH), jnp.int32),
             pltpu.VMEM((ECH, NF), jnp.float32),
             pltpu.VMEM((ECH, NF), jnp.float32),
             pltpu.VMEM_SHARED((HALF, NF), jnp.float32),
             pltpu.VMEM_SHARED((ACC_R, NF), jnp.float32)]
            + [pltpu.SemaphoreType.DMA] * 3
        ),
    )


def _sc_segsum(src, dst, xw1, zeros):
    return _sc_segsum_fn()(src, dst, xw1, zeros)


def _edgemap_body(src_ref, dst_ref, sm_ref, dm_ref):
    sv = src_ref[...]
    dv = dst_ref[...]
    in0 = sv < HALF
    sm_ref[0] = jnp.where(in0, sv, 0)
    sm_ref[1] = jnp.where(in0, 0, sv - HALF)
    dm_ref[0] = jnp.where(in0, dv, DUMP)
    dm_ref[1] = jnp.where(in0, DUMP, dv)


def _edgemap(src, dst):
    r = E_PAD // 128
    return pl.pallas_call(
        _edgemap_body,
        out_shape=[jax.ShapeDtypeStruct((2, r, 128), jnp.int32),
                   jax.ShapeDtypeStruct((2, r, 128), jnp.int32)],
    )(src.reshape(r, 128), dst.reshape(r, 128))


def kernel(vertex_features, vertex_positions, img_feat0, img_feat1,
           img_feat2, img_feat3, lin0_w, gc0_w0, gc0_w1, gc1_w0, gc1_w1,
           gc2_w0, gc2_w1, lin1_w, vertex_adjacency):
    f32 = jnp.float32
    pad_n = N_PAD - N

    pos = jnp.pad(vertex_positions, ((0, pad_n), (0, 0)))
    vfeat = jnp.pad(vertex_features, ((0, pad_n), (0, 0)))

    r = N_PAD // 128
    px = pos[:, 0].reshape(r, 128)
    py = pos[:, 1].reshape(r, 128)
    pz = pos[:, 2].reshape(r, 128)
    idx = _coords(px, py, pz).reshape(3, 2 * N_PAD)

    fms = [img_feat0, img_feat1, img_feat2, img_feat3]
    ptables = []
    off = 0
    for m, (c, s) in enumerate(FEAT):
        w_m = lin0_w[off:off + c]
        off += c
        ptables.append(_ptable(fms[m].reshape(B, c, s * s), w_m, s * s))

    p0 = jnp.pad(ptables[0], ((0, 16), (0, 0)))
    p1 = jnp.pad(ptables[1], ((0, 16), (0, 0)))
    p2 = jnp.pad(ptables[2], ((0, 16), (0, 0)))
    p3 = jnp.pad(ptables[3], ((0, 8), (0, 0)))
    stg0 = jnp.concatenate([p0[:M0H], p1], axis=0)
    stg1 = jnp.concatenate([p0[M0H:], p2, p3], axis=0)
    stg = jnp.stack([jnp.pad(stg0, ((0, STG_R - 9432), (0, 0))),
                     jnp.pad(stg1, ((0, STG_R - 7284), (0, 0)))])

    proj = _sc_align(idx, stg)

    src = jnp.concatenate([vertex_adjacency[0],
                           jnp.full((E_PAD - E,), N, jnp.int32)])
    dst = jnp.concatenate([vertex_adjacency[1],
                           jnp.zeros((E_PAD - E,), jnp.int32)])
    srcm, dstm = _edgemap(src, dst)
    srcm = srcm.reshape(2 * 16 * NSEG, 4, 128)
    dstm = dstm.reshape(2 * 16 * NSEG, CPS, ECH)
    zeros = jnp.zeros((ACC_PT, NF), f32)

    xw0, xw1 = _conv_in(vfeat, pos, proj, gc0_w0, gc0_w1)
    part = _sc_segsum(srcm, dstm, xw1, zeros)
    xw0, xw1 = _conv_mid(pos, xw0, part, gc1_w0, gc1_w1)
    part = _sc_segsum(srcm, dstm, xw1, zeros)
    xw0, xw1 = _conv_mid(pos, xw0, part, gc2_w0, gc2_w1)
    part = _sc_segsum(srcm, dstm, xw1, zeros)
    nf, new_pos = _final(pos, xw0, part, lin1_w)

    return (new_pos[:N], nf[:N])


# speedup vs baseline: 9.8449x; 1.0295x over previous
import functools

import jax
import jax.numpy as jnp
from jax import lax
from jax.experimental import pallas as pl
from jax.experimental.pallas import tpu as pltpu
from jax.experimental.pallas import tpu_sc as plsc

B = 4
NV = 2466
N = B * NV
E = 59184
NF = 128
NDIMS = 3
IMG_HW = 224

FEAT = [(256, 56), (512, 28), (1024, 14), (2048, 7)]
HWS = [s * s for _, s in FEAT]

N_PAD = 9984
VPT = 624
VCH = 104
NVCH = VPT // VCH

E_PAD = 65536
HALF = 4992
ECH = 32
CPS = 16
NSEG = 8
ACC_R = 9992
DUMP = N

ROWS_PT = N_PAD // 32
ACC_PT = N_PAD // 16

M0H = 6280
STG_R = 9472
STG_PT = STG_R // 16
ZR0 = M0H + 3144
ZR1 = M0H + 792

@functools.lru_cache(maxsize=None)
def _mesh():
    return plsc.VectorSubcoreMesh(
        core_axis_name="c", subcore_axis_name="s",
        num_cores=2, num_subcores=16)


def _dot(a, b):
    return lax.dot_general(a, b, (((1,), (0,)), ((), ())),
                           preferred_element_type=jnp.float32)


def _coords_body(px_ref, py_ref, pz_ref, out_ref):
    pxv = px_ref[...]
    pyv = py_ref[...]
    pzv = pz_ref[...]
    rows = lax.broadcasted_iota(jnp.int32, (N_PAD // 128, 128), 0)
    cols = lax.broadcasted_iota(jnp.int32, (N_PAD // 128, 128), 1)
    n = rows * 128 + cols
    valid_n = n < N
    pz_safe = jnp.where(valid_n, pzv, 1.0)
    h = 248.0 * (pyv / pz_safe) + 111.5
    w = 248.0 * (pxv / (-pz_safe)) + 111.5
    h = jnp.clip(h, 0.0, IMG_HW - 1.0)
    w = jnp.clip(w, 0.0, IMG_HW - 1.0)
    bidx = n // NV
    gidx = []
    for m, (_, s) in enumerate(FEAT):
        x = w / (float(IMG_HW) / s)
        y = h / (float(IMG_HW) / s)
        x1 = jnp.floor(x).astype(jnp.int32)
        x2 = jnp.minimum(jnp.ceil(x), s - 1).astype(jnp.int32)
        y1 = jnp.floor(y).astype(jnp.int32)
        y2 = jnp.minimum(jnp.ceil(y), s - 1).astype(jnp.int32)
        ok = (x2 > x1) & (y2 > y1) & valid_n
        gidx.append((ok, bidx * (s * s) + x1 * s + y1))
    ok0, g0 = gidx[0]
    ok1, g1 = gidx[1]
    ok2, g2 = gidx[2]
    ok3, g3 = gidx[3]
    out_ref[0, 0] = jnp.where(ok0 & (g0 < M0H), g0, ZR0)
    out_ref[0, 1] = jnp.where(ok0 & (g0 >= M0H), g0 - M0H, ZR1)
    out_ref[1, 0] = jnp.where(ok1, M0H + g1, ZR0)
    out_ref[1, 1] = jnp.where(ok2, M0H + g2, ZR1)
    out_ref[2, 0] = jnp.full_like(g0, ZR0)
    out_ref[2, 1] = jnp.where(ok3, M0H + 800 + g3, ZR1)


def _coords(px, py, pz):
    r = N_PAD // 128
    return pl.pallas_call(
        _coords_body,
        out_shape=jax.ShapeDtypeStruct((3, 2, r, 128), jnp.int32),
    )(px, py, pz)


def _ptable_body(fm_ref, w_ref, out_ref):
    out_ref[0] = lax.dot_general(
        fm_ref[0], w_ref[...], (((0,), (0,)), ((), ())),
        preferred_element_type=jnp.float32)


def _ptable(fm, w, hw):
    c = fm.shape[1]
    out = pl.pallas_call(
        _ptable_body,
        grid=(B,),
        in_specs=[
            pl.BlockSpec((1, c, hw), lambda b: (b, 0, 0)),
            pl.BlockSpec((c, NF), lambda b: (0, 0)),
        ],
        out_specs=pl.BlockSpec((1, hw, NF), lambda b: (b, 0, 0)),
        out_shape=jax.ShapeDtypeStruct((B, hw, NF), jnp.float32),
    )(fm, w)
    return out.reshape(B * hw, NF)


_MBLK = 2496


def _conv_in_body(vf_ref, pos_ref, pj_ref,
                  w0a, w0b, w0c, w1a, w1b, w1c, o0_ref, o1_ref):
    vf = vf_ref[...]
    ps = pos_ref[...]
    pj = pj_ref[0] + pj_ref[1]
    o0_ref[...] = _dot(vf, w0a[...]) + _dot(ps, w0b[...]) + _dot(pj, w0c[...])
    o1_ref[...] = _dot(vf, w1a[...]) + _dot(ps, w1b[...]) + _dot(pj, w1c[...])


def _conv_in(vfeat, pos, pj, w0, w1):
    w0a, w0b, w0c = w0[:NF], w0[NF:NF + NDIMS], w0[NF + NDIMS:]
    w1a, w1b, w1c = w1[:NF], w1[NF:NF + NDIMS], w1[NF + NDIMS:]
    g = N_PAD // _MBLK
    row = lambda i: (i, 0)
    full = lambda i: (0, 0)
    return pl.pallas_call(
        _conv_in_body,
        grid=(g,),
        in_specs=[
            pl.BlockSpec((_MBLK, NF), row),
            pl.BlockSpec((_MBLK, NDIMS), row),
            pl.BlockSpec((2, _MBLK, NF), lambda i: (0, i, 0)),
            pl.BlockSpec((NF, NF), full),
            pl.BlockSpec((NDIMS, NF), full),
            pl.BlockSpec((NF, NF), full),
            pl.BlockSpec((NF, NF), full),
            pl.BlockSpec((NDIMS, NF), full),
            pl.BlockSpec((NF, NF), full),
        ],
        out_specs=[pl.BlockSpec((_MBLK, NF), row),
                   pl.BlockSpec((_MBLK, NF), row)],
        out_shape=[jax.ShapeDtypeStruct((N_PAD, NF), jnp.float32),
                   jax.ShapeDtypeStruct((N_PAD, NF), jnp.float32)],
    )(vfeat, pos, pj, w0a, w0b, w0c, w1a, w1b, w1c)


def _conv_mid_body(pos_ref, a_ref, part_ref,
                   w0a, w0b, w1a, w1b, o0_ref, o1_ref):
    ps = pos_ref[...]
    h = jnp.maximum(a_ref[...] + part_ref[0] + part_ref[1], 0.0)
    o0_ref[...] = _dot(ps, w0a[...]) + _dot(h, w0b[...])
    o1_ref[...] = _dot(ps, w1a[...]) + _dot(h, w1b[...])


def _conv_mid(pos, a, partials, w0, w1):
    w0a, w0b = w0[:NDIMS], w0[NDIMS:]
    w1a, w1b = w1[:NDIMS], w1[NDIMS:]
    g = N_PAD // _MBLK
    row = lambda i: (i, 0)
    full = lambda i: (0, 0)
    return pl.pallas_call(
        _conv_mid_body,
        grid=(g,),
        in_specs=[
            pl.BlockSpec((_MBLK, NDIMS), row),
            pl.BlockSpec((_MBLK, NF), row),
            pl.BlockSpec((2, _MBLK, NF), lambda i: (0, i, 0)),
            pl.BlockSpec((NDIMS, NF), full),
            pl.BlockSpec((NF, NF), full),
            pl.BlockSpec((NDIMS, NF), full),
            pl.BlockSpec((NF, NF), full),
        ],
        out_specs=[pl.BlockSpec((_MBLK, NF), row),
                   pl.BlockSpec((_MBLK, NF), row)],
        out_shape=[jax.ShapeDtypeStruct((N_PAD, NF), jnp.float32),
                   jax.ShapeDtypeStruct((N_PAD, NF), jnp.float32)],
    )(pos, a, partials, w0a, w0b, w1a, w1b)


def _final_body(pos_ref, a_ref, part_ref, lw_ref, nf_ref, np_ref):
    nf = jnp.maximum(a_ref[...] + part_ref[0] + part_ref[1], 0.0)
    nf_ref[...] = nf
    np_ref[...] = pos_ref[...] + jnp.tanh(_dot(nf, lw_ref[...]))


def _final(pos, a, partials, lin1_w):
    g = N_PAD // _MBLK
    row = lambda i: (i, 0)
    return pl.pallas_call(
        _final_body,
        grid=(g,),
        in_specs=[
            pl.BlockSpec((_MBLK, NDIMS), row),
            pl.BlockSpec((_MBLK, NF), row),
            pl.BlockSpec((2, _MBLK, NF), lambda i: (0, i, 0)),
            pl.BlockSpec((NF, NDIMS), lambda i: (0, 0)),
        ],
        out_specs=[pl.BlockSpec((_MBLK, NF), row),
                   pl.BlockSpec((_MBLK, NDIMS), row)],
        out_shape=[jax.ShapeDtypeStruct((N_PAD, NF), jnp.float32),
                   jax.ShapeDtypeStruct((N_PAD, NDIMS), jnp.float32)],
    )(pos, a, partials, lin1_w)


def _align_body(i0, i1, i2, stg_hbm, out_hbm,
                iv0, iv1, iv2, g0, g1, g2, stgv,
                gs0, gs1, gs2, ssem):
    c = lax.axis_index("c")
    s = lax.axis_index("s")
    base = s * VPT
    gs = (g0, g1, g2)
    gsem = (gs0, gs1, gs2)
    idxv = (iv0, iv1, iv2)
    scp = pltpu.async_copy(stg_hbm.at[c, pl.ds(s * STG_PT, STG_PT)],
                           stgv.at[pl.ds(s * STG_PT, STG_PT)], ssem)
    for m, im in enumerate((i0, i1, i2)):
        pltpu.sync_copy(im.at[pl.ds(c * N_PAD + base, VPT)], idxv[m])
    scp.wait()
    plsc.subcore_barrier()
    for ch in range(NVCH):
        off = base + ch * VCH
        cps = [pltpu.async_copy(
                   stgv.at[idxv[m].at[pl.ds(ch * VCH, VCH)]], gs[m], gsem[m])
               for m in range(3)]
        for cp in cps:
            cp.wait()

        def body(i, _):
            for k in range(NF // 16):
                sl = pl.ds(k * 16, 16)
                g0[i, sl] = g0[i, sl] + g1[i, sl] + g2[i, sl]
            return 0

        lax.fori_loop(0, VCH, body, 0, unroll=2)
        pltpu.sync_copy(g0, out_hbm.at[c, pl.ds(off, VCH)])


@functools.lru_cache(maxsize=None)
def _sc_align_fn():
    return pl.kernel(
        _align_body,
        out_type=jax.ShapeDtypeStruct((2, N_PAD, NF), jnp.float32),
        mesh=_mesh(),
        scratch_types=(
            [pltpu.VMEM((VPT,), jnp.int32)] * 3
            + [pltpu.VMEM((VCH, NF), jnp.float32)] * 3
            + [pltpu.VMEM_SHARED((STG_R, NF), jnp.float32)]
            + [pltpu.SemaphoreType.DMA] * 4
        ),
    )


def _sc_align(idx, stg):
    return _sc_align_fn()(idx[0], idx[1], idx[2], stg)


def _seg_body(srcm_hbm, dstm_hbm, xw1_hbm, zeros_hbm, out_hbm,
              srcv, dstv, g0, g1, xw1s, acc, gs0, gs1, ssem):
    c = lax.axis_index("c")
    s = lax.axis_index("s")
    gs = (g0, g1)
    gsem = (gs0, gs1)
    scp = pltpu.async_copy(xw1_hbm.at[pl.ds(c * HALF + s * ROWS_PT, ROWS_PT)],
                           xw1s.at[pl.ds(s * ROWS_PT, ROWS_PT)], ssem)
    @pl.when(s < 15)
    def _():
        pltpu.sync_copy(zeros_hbm.at[pl.ds(0, ACC_PT)],
                        acc.at[pl.ds(s * ACC_PT, ACC_PT)])

    @pl.when(s == 15)
    def _():
        pltpu.sync_copy(zeros_hbm.at[pl.ds(0, ACC_R - 15 * ACC_PT)],
                        acc.at[pl.ds(15 * ACC_PT, ACC_R - 15 * ACC_PT)])

    scp.wait()
    plsc.subcore_barrier()
    for t in range(NSEG):
        seg = (c * 16 + s) * NSEG + t
        pltpu.sync_copy(srcm_hbm.at[seg], srcv)
        pltpu.sync_copy(dstm_hbm.at[seg], dstv)

        def fire(j, b):
            idx = srcv.at[j // 4, pl.ds((j % 4) * ECH, ECH)]
            return pltpu.async_copy(xw1s.at[idx], gs[b], gsem[b])

        gcp = [fire(j, j) for j in range(2)]
        for j in range(CPS):
            b = j % 2
            gcp[b].wait()
            pltpu.sync_copy(gs[b], acc.at[dstv.at[j]], add=True)
            if j + 2 < CPS:
                gcp[b] = fire(j + 2, b)
    plsc.subcore_barrier()
    @pl.when(s < 15)
    def _():
        pltpu.sync_copy(acc.at[pl.ds(s * ACC_PT, ACC_PT)],
                        out_hbm.at[c, pl.ds(s * ACC_PT, ACC_PT)])

    @pl.when(s == 15)
    def _():
        pltpu.sync_copy(acc.at[pl.ds(15 * ACC_PT, N - 15 * ACC_PT)],
                        out_hbm.at[c, pl.ds(15 * ACC_PT, N - 15 * ACC_PT)])
        pltpu.sync_copy(zeros_hbm.at[pl.ds(0, N_PAD - N)],
                        out_hbm.at[c, pl.ds(N, N_PAD - N)])


@functools.lru_cache(maxsize=None)
def _sc_segsum_fn():
    return pl.kernel(
        _seg_body,
        out_type=jax.ShapeDtypeStruct((2, N_PAD, NF), jnp.float32),
        mesh=_mesh(),
        scratch_types=(
            [pltpu.VMEM((4, 128), jnp.int32),
             pltpu.VMEM((CPS, ECH), jnp.int32),
             pltpu.VMEM((ECH, NF), jnp.float32),
             pltpu.VMEM((ECH, NF), jnp.float32),
             pltpu.VMEM_SHARED((HALF, NF), jnp.float32),
             pltpu.VMEM_SHARED((ACC_R, NF), jnp.float32)]
            + [pltpu.SemaphoreType.DMA] * 3
        ),
    )


def _sc_segsum(src, dst, xw1, zeros):
    return _sc_segsum_fn()(src, dst, xw1, zeros)


def _edgemap_body(src_ref, dst_ref, sm_ref, dm_ref):
    sv = src_ref[...]
    dv = dst_ref[...]
    in0 = sv < HALF
    r = E_PAD // 128
    rows = lax.broadcasted_iota(jnp.int32, (r, 128), 0)
    cols = lax.broadcasted_iota(jnp.int32, (r, 128), 1)
    dump = DUMP + (rows // 32) * 8 + (cols % 8)
    sm_ref[0] = jnp.where(in0, sv, 0)
    sm_ref[1] = jnp.where(in0, 0, sv - HALF)
    dm_ref[0] = jnp.where(in0, dv, dump)
    dm_ref[1] = jnp.where(in0, dump, dv)


def _edgemap(src, dst):
    r = E_PAD // 128
    return pl.pallas_call(
        _edgemap_body,
        out_shape=[jax.ShapeDtypeStruct((2, r, 128), jnp.int32),
                   jax.ShapeDtypeStruct((2, r, 128), jnp.int32)],
    )(src.reshape(r, 128), dst.reshape(r, 128))


def kernel(vertex_features, vertex_positions, img_feat0, img_feat1,
           img_feat2, img_feat3, lin0_w, gc0_w0, gc0_w1, gc1_w0, gc1_w1,
           gc2_w0, gc2_w1, lin1_w, vertex_adjacency):
    f32 = jnp.float32
    pad_n = N_PAD - N

    pos = jnp.pad(vertex_positions, ((0, pad_n), (0, 0)))
    vfeat = jnp.pad(vertex_features, ((0, pad_n), (0, 0)))

    r = N_PAD // 128
    px = pos[:, 0].reshape(r, 128)
    py = pos[:, 1].reshape(r, 128)
    pz = pos[:, 2].reshape(r, 128)
    idx = _coords(px, py, pz).reshape(3, 2 * N_PAD)

    fms = [img_feat0, img_feat1, img_feat2, img_feat3]
    ptables = []
    off = 0
    for m, (c, s) in enumerate(FEAT):
        w_m = lin0_w[off:off + c]
        off += c
        ptables.append(_ptable(fms[m].reshape(B, c, s * s), w_m, s * s))

    p0 = jnp.pad(ptables[0], ((0, 16), (0, 0)))
    p1 = jnp.pad(ptables[1], ((0, 16), (0, 0)))
    p2 = jnp.pad(ptables[2], ((0, 16), (0, 0)))
    p3 = jnp.pad(ptables[3], ((0, 8), (0, 0)))
    stg0 = jnp.concatenate([p0[:M0H], p1], axis=0)
    stg1 = jnp.concatenate([p0[M0H:], p2, p3], axis=0)
    stg = jnp.stack([jnp.pad(stg0, ((0, STG_R - 9432), (0, 0))),
                     jnp.pad(stg1, ((0, STG_R - 7284), (0, 0)))])

    proj = _sc_align(idx, stg)

    src = jnp.concatenate([vertex_adjacency[0],
                           jnp.full((E_PAD - E,), N, jnp.int32)])
    dst = jnp.concatenate([vertex_adjacency[1],
                           jnp.zeros((E_PAD - E,), jnp.int32)])
    srcm, dstm = _edgemap(src, dst)
    srcm = srcm.reshape(2 * 16 * NSEG, 4, 128)
    dstm = dstm.reshape(2 * 16 * NSEG, CPS, ECH)
    zeros = jnp.zeros((ACC_R - 15 * ACC_PT, NF), f32)

    xw0, xw1 = _conv_in(vfeat, pos, proj, gc0_w0, gc0_w1)
    part = _sc_segsum(srcm, dstm, xw1, zeros)
    xw0, xw1 = _conv_mid(pos, xw0, part, gc1_w0, gc1_w1)
    part = _sc_segsum(srcm, dstm, xw1, zeros)
    xw0, xw1 = _conv_mid(pos, xw0, part, gc2_w0, gc2_w1)
    part = _sc_segsum(srcm, dstm, xw1, zeros)
    nf, new_pos = _final(pos, xw0, part, lin1_w)

    return (new_pos[:N], nf[:N])
